# baseline (speedup 1.0000x reference)
import functools

import jax
import jax.numpy as jnp
from jax import lax
from jax.experimental import pallas as pl
from jax.experimental.pallas import tpu as pltpu
from jax.experimental.pallas import tpu_sc as plsc

N_NODES = 10000
N_EDGES = 160000
D_IN = 256
D_HID = 256
N_CLASSES = 47

NC = 2
NS = 16
L = 16

E_PAD = 163840
EPS = E_PAD // NS
BLK = 1024
NB = EPS // BLK
RND_ROWS = 2
N_ACC = 10016
ZR = 624
ZTAIL = N_NODES - NS * ZR
DH = 128


def _mesh():
    return plsc.VectorSubcoreMesh(
        core_axis_name="c", subcore_axis_name="s",
        num_cores=NC, num_subcores=NS,
    )


def _stripe_copy(sid, src, dst):
    zbase = sid * ZR
    pltpu.sync_copy(src.at[pl.ds(zbase, ZR)], dst.at[pl.ds(zbase, ZR)])

    @pl.when(sid == NS - 1)
    def _():
        pltpu.sync_copy(src.at[pl.ds(NS * ZR, ZTAIL)],
                        dst.at[pl.ds(NS * ZR, ZTAIL)])


GSUB = 4
GROWS = 128 // GSUB


def _fire_gather(h_ref, src_v, rows_v, j, buf, gsem):
    return [
        pltpu.async_copy(
            h_ref.at[src_v.at[j, pl.ds(k * GROWS, GROWS)]],
            rows_v.at[buf, pl.ds(k * GROWS, GROWS)],
            gsem,
        )
        for k in range(GSUB)
    ]


def _sc_agg_body(h0, h1, srcp, dstp, z128, s0_out, s1_out,
                 acc, src_v, dst_v, rows_v, gsem, ssem0, ssem1):
    cid = lax.axis_index("c")
    sid = lax.axis_index("s")
    ssems = (ssem0, ssem1)

    _stripe_copy(sid, z128, acc)
    plsc.subcore_barrier()

    def do_edges(h_ref):
        def blk_body(i, carry):
            blk = sid * NB + i
            pltpu.sync_copy(srcp.at[blk], src_v)
            pltpu.sync_copy(dstp.at[blk], dst_v)
            gd = _fire_gather(h_ref, src_v, rows_v, 0, 0, gsem)
            scat = [None] * 8
            for j in range(8):
                for g in gd:
                    g.wait()
                scat[j] = pltpu.async_copy(
                    rows_v.at[j % 2],
                    acc.at[dst_v.at[j]],
                    ssems[j % 2],
                    add=True,
                )
                if j < 7:
                    if j >= 1:
                        scat[j - 1].wait()
                    gd = _fire_gather(h_ref, src_v, rows_v, j + 1,
                                      (j + 1) % 2, gsem)
            scat[6].wait()
            scat[7].wait()
            return carry
        lax.fori_loop(0, NB, blk_body, 0)

    @pl.when(cid == 0)
    def _():
        do_edges(h0)

    @pl.when(cid == 1)
    def _():
        do_edges(h1)

    plsc.subcore_barrier()

    @pl.when(cid == 0)
    def _():
        _stripe_copy(sid, acc, s0_out)

    @pl.when(cid == 1)
    def _():
        _stripe_copy(sid, acc, s1_out)


def _make_sc_agg():
    f32 = jnp.float32
    return pl.kernel(
        _sc_agg_body,
        out_type=[
            jax.ShapeDtypeStruct((N_NODES, DH), f32),
            jax.ShapeDtypeStruct((N_NODES, DH), f32),
        ],
        mesh=_mesh(),
        scratch_types=[
            pltpu.VMEM_SHARED((N_ACC, DH), f32),
            pltpu.VMEM((8, 128), jnp.int32),
            pltpu.VMEM((8, 128), jnp.int32),
            pltpu.VMEM((2, 128, DH), f32),
            pltpu.SemaphoreType.DMA,
            pltpu.SemaphoreType.DMA,
            pltpu.SemaphoreType.DMA,
        ],
    )


NBH = NB // NC


def _sc_deg_body(dstp, z128, ones_hbm, d0_out, d1_out, dega, dst_v, ones_v,
                 sem):
    cid = lax.axis_index("c")
    sid = lax.axis_index("s")

    _stripe_copy(sid, z128, dega)
    pltpu.sync_copy(ones_hbm, ones_v)
    plsc.subcore_barrier()

    def blk_body(i, carry):
        blk = (sid * NC + cid) * NBH + i
        pltpu.sync_copy(dstp.at[blk], dst_v)
        for j in range(8):
            pltpu.sync_copy(ones_v, dega.at[dst_v.at[j]], add=True)
        return carry
    lax.fori_loop(0, NBH, blk_body, 0)

    plsc.subcore_barrier()

    @pl.when(cid == 0)
    def _():
        _stripe_copy(sid, dega, d0_out)

    @pl.when(cid == 1)
    def _():
        _stripe_copy(sid, dega, d1_out)


def _make_sc_deg():
    f32 = jnp.float32
    return pl.kernel(
        _sc_deg_body,
        out_type=[
            jax.ShapeDtypeStruct((N_NODES, DH), f32),
            jax.ShapeDtypeStruct((N_NODES, DH), f32),
        ],
        mesh=_mesh(),
        scratch_types=[
            pltpu.VMEM_SHARED((N_ACC, DH), f32),
            pltpu.VMEM((8, 128), jnp.int32),
            pltpu.VMEM((128, DH), f32),
            pltpu.SemaphoreType.DMA,
        ],
    )


def _sc_agg_split_body(y, srcp, dstp, z128, p0_out, p1_out,
                       acc, src_v, dst_v, rows_v, gsem, ssem0, ssem1):
    cid = lax.axis_index("c")
    sid = lax.axis_index("s")
    ssems = (ssem0, ssem1)

    _stripe_copy(sid, z128, acc)
    plsc.subcore_barrier()

    def blk_body(i, carry):
        blk = (sid * NC + cid) * NBH + i
        pltpu.sync_copy(srcp.at[blk], src_v)
        pltpu.sync_copy(dstp.at[blk], dst_v)
        gd = _fire_gather(y, src_v, rows_v, 0, 0, gsem)
        scat = [None] * 8
        for j in range(8):
            for g in gd:
                g.wait()
            scat[j] = pltpu.async_copy(
                rows_v.at[j % 2], acc.at[dst_v.at[j]], ssems[j % 2],
                add=True,
            )
            if j < 7:
                if j >= 1:
                    scat[j - 1].wait()
                gd = _fire_gather(y, src_v, rows_v, j + 1, (j + 1) % 2,
                                  gsem)
        scat[6].wait()
        scat[7].wait()
        return carry
    lax.fori_loop(0, NBH, blk_body, 0)

    plsc.subcore_barrier()

    @pl.when(cid == 0)
    def _():
        _stripe_copy(sid, acc, p0_out)

    @pl.when(cid == 1)
    def _():
        _stripe_copy(sid, acc, p1_out)


def _make_sc_agg_split():
    f32 = jnp.float32
    return pl.kernel(
        _sc_agg_split_body,
        out_type=[
            jax.ShapeDtypeStruct((N_NODES, DH), f32),
            jax.ShapeDtypeStruct((N_NODES, DH), f32),
        ],
        mesh=_mesh(),
        scratch_types=[
            pltpu.VMEM_SHARED((N_ACC, DH), f32),
            pltpu.VMEM((8, 128), jnp.int32),
            pltpu.VMEM((8, 128), jnp.int32),
            pltpu.VMEM((2, 128, DH), f32),
            pltpu.SemaphoreType.DMA,
            pltpu.SemaphoreType.DMA,
            pltpu.SemaphoreType.DMA,
        ],
    )


def _dot(a, b):
    return jnp.dot(a, b, preferred_element_type=jnp.float32)


def _sage_block(h0, h1, s0, s1, d0, d1, ws, wn):
    h = jnp.concatenate([h0[...], h1[...]], axis=1)
    s = jnp.concatenate([s0[...], s1[...]], axis=1)
    d = jnp.maximum(d0[:, 0:1] + d1[:, 0:1], 1.0)
    return _dot(h, ws[...]) + _dot(s / d, wn[...])


def _tc_layer0_body(h0, h1, s0, s1, d0, d1, ws, wn, o0, o1):
    out = jnp.maximum(_sage_block(h0, h1, s0, s1, d0, d1, ws, wn), 0.0)
    o0[...] = out[:, :DH]
    o1[...] = out[:, DH:]


def _tc_layer1_body(h0, h1, s0, s1, d0, d1, ws, wn, ws2, wn2, y2, z2):
    h2 = jnp.maximum(_sage_block(h0, h1, s0, s1, d0, d1, ws, wn), 0.0)
    y2[...] = _dot(h2, wn2[...])
    z2[...] = _dot(h2, ws2[...])


def _tc_final_body(z2, p0, p1, d0, d1, o):
    d = jnp.maximum(d0[:, 0:1] + d1[:, 0:1], 1.0)
    o[...] = z2[...] + (p0[...] + p1[...]) / d


BR = 400


def _row_spec(w):
    return pl.BlockSpec((BR, w), lambda i: (i, 0))


def _full_spec(a, b):
    return pl.BlockSpec((a, b), lambda i: (0, 0))


def _tc_layer0(h0, h1, s0, s1, d0, d1, ws_t, wn_t):
    return pl.pallas_call(
        _tc_layer0_body,
        grid=(N_NODES // BR,),
        in_specs=[_row_spec(DH)] * 6 + [
            _full_spec(D_HID, D_HID), _full_spec(D_HID, D_HID),
        ],
        out_specs=[_row_spec(DH), _row_spec(DH)],
        out_shape=[
            jax.ShapeDtypeStruct((N_NODES, DH), jnp.float32),
            jax.ShapeDtypeStruct((N_NODES, DH), jnp.float32),
        ],
    )(h0, h1, s0, s1, d0, d1, ws_t, wn_t)


def _tc_layer1(h0, h1, s0, s1, d0, d1, ws_t, wn_t, ws2_t, wn2_t):
    return pl.pallas_call(
        _tc_layer1_body,
        grid=(N_NODES // BR,),
        in_specs=[_row_spec(DH)] * 6 + [
            _full_spec(D_HID, D_HID), _full_spec(D_HID, D_HID),
            _full_spec(D_HID, DH), _full_spec(D_HID, DH),
        ],
        out_specs=[_row_spec(DH), _row_spec(DH)],
        out_shape=[
            jax.ShapeDtypeStruct((N_NODES, DH), jnp.float32),
            jax.ShapeDtypeStruct((N_NODES, DH), jnp.float32),
        ],
    )(h0, h1, s0, s1, d0, d1, ws_t, wn_t, ws2_t, wn2_t)


def _tc_final(z2, p0, p1, d0, d1):
    return pl.pallas_call(
        _tc_final_body,
        grid=(N_NODES // BR,),
        in_specs=[_row_spec(DH)] * 5,
        out_specs=_row_spec(DH),
        out_shape=jax.ShapeDtypeStruct((N_NODES, DH), jnp.float32),
    )(z2, p0, p1, d0, d1)


def kernel(x, edge_index, W_self_0, W_neigh_0, W_self_1, W_neigh_1,
           W_self_2, W_neigh_2):
    f32 = jnp.float32
    x0 = x[:, :DH]
    x1 = x[:, DH:]
    src = edge_index[0]
    dst = edge_index[1]
    pad = E_PAD - N_EDGES
    srcp = jnp.concatenate([src, jnp.zeros((pad,), jnp.int32)]).reshape(
        E_PAD // BLK, 8, 128)
    dstp = jnp.concatenate([dst, jnp.full((pad,), N_NODES, jnp.int32)]
                           ).reshape(E_PAD // BLK, 8, 128)
    z128 = jnp.zeros((N_NODES, DH), f32)
    ones = jnp.ones((128, DH), f32)

    sc_agg = _make_sc_agg()
    d0, d1 = _make_sc_deg()(dstp, z128, ones)

    w_pad = 128 - N_CLASSES
    ws2 = jnp.pad(W_self_2.T, ((0, 0), (0, w_pad)))
    wn2 = jnp.pad(W_neigh_2.T, ((0, 0), (0, w_pad)))

    s0, s1 = sc_agg(x0, x1, srcp, dstp, z128)
    h0, h1 = _tc_layer0(x0, x1, s0, s1, d0, d1, W_self_0.T, W_neigh_0.T)
    s0, s1 = sc_agg(h0, h1, srcp, dstp, z128)
    y2, z2 = _tc_layer1(h0, h1, s0, s1, d0, d1, W_self_1.T, W_neigh_1.T,
                        ws2, wn2)
    p0, p1 = _make_sc_agg_split()(y2, srcp, dstp, z128)
    out = _tc_final(z2, p0, p1, d0, d1)
    return out[:, :N_CLASSES]


# speedup vs baseline: 1.0050x; 1.0050x over previous
import functools

import jax
import jax.numpy as jnp
from jax import lax
from jax.experimental import pallas as pl
from jax.experimental.pallas import tpu as pltpu
from jax.experimental.pallas import tpu_sc as plsc

N_NODES = 10000
N_EDGES = 160000
D_IN = 256
D_HID = 256
N_CLASSES = 47

NC = 2
NS = 16
L = 16

E_PAD = 163840
EPS = E_PAD // NS
BLK = 1024
NB = EPS // BLK
RND_ROWS = 2
N_ACC = 10016
ZR = 624
ZTAIL = N_NODES - NS * ZR
DH = 128


def _mesh():
    return plsc.VectorSubcoreMesh(
        core_axis_name="c", subcore_axis_name="s",
        num_cores=NC, num_subcores=NS,
    )


def _stripe_copy(sid, src, dst):
    zbase = sid * ZR
    pltpu.sync_copy(src.at[pl.ds(zbase, ZR)], dst.at[pl.ds(zbase, ZR)])

    @pl.when(sid == NS - 1)
    def _():
        pltpu.sync_copy(src.at[pl.ds(NS * ZR, ZTAIL)],
                        dst.at[pl.ds(NS * ZR, ZTAIL)])


GSUB = 2
GROWS = 128 // GSUB


def _fire_gather(h_ref, src_v, rows_v, j, buf, gsem):
    return [
        pltpu.async_copy(
            h_ref.at[src_v.at[j, pl.ds(k * GROWS, GROWS)]],
            rows_v.at[buf, pl.ds(k * GROWS, GROWS)],
            gsem,
        )
        for k in range(GSUB)
    ]


def _sc_agg_body(h0, h1, srcp, dstp, z128, s0_out, s1_out,
                 acc, src_v, dst_v, rows_v, gsem, ssem0, ssem1):
    cid = lax.axis_index("c")
    sid = lax.axis_index("s")
    ssems = (ssem0, ssem1)

    _stripe_copy(sid, z128, acc)
    plsc.subcore_barrier()

    def do_edges(h_ref):
        def blk_body(i, carry):
            blk = sid * NB + i
            pltpu.sync_copy(srcp.at[blk], src_v)
            pltpu.sync_copy(dstp.at[blk], dst_v)
            gd = _fire_gather(h_ref, src_v, rows_v, 0, 0, gsem)
            scat = [None] * 8
            for j in range(8):
                for g in gd:
                    g.wait()
                scat[j] = pltpu.async_copy(
                    rows_v.at[j % 2],
                    acc.at[dst_v.at[j]],
                    ssems[j % 2],
                    add=True,
                )
                if j < 7:
                    if j >= 1:
                        scat[j - 1].wait()
                    gd = _fire_gather(h_ref, src_v, rows_v, j + 1,
                                      (j + 1) % 2, gsem)
            scat[6].wait()
            scat[7].wait()
            return carry
        lax.fori_loop(0, NB, blk_body, 0)

    @pl.when(cid == 0)
    def _():
        do_edges(h0)

    @pl.when(cid == 1)
    def _():
        do_edges(h1)

    plsc.subcore_barrier()

    @pl.when(cid == 0)
    def _():
        _stripe_copy(sid, acc, s0_out)

    @pl.when(cid == 1)
    def _():
        _stripe_copy(sid, acc, s1_out)


def _make_sc_agg():
    f32 = jnp.float32
    return pl.kernel(
        _sc_agg_body,
        out_type=[
            jax.ShapeDtypeStruct((N_NODES, DH), f32),
            jax.ShapeDtypeStruct((N_NODES, DH), f32),
        ],
        mesh=_mesh(),
        scratch_types=[
            pltpu.VMEM_SHARED((N_ACC, DH), f32),
            pltpu.VMEM((8, 128), jnp.int32),
            pltpu.VMEM((8, 128), jnp.int32),
            pltpu.VMEM((2, 128, DH), f32),
            pltpu.SemaphoreType.DMA,
            pltpu.SemaphoreType.DMA,
            pltpu.SemaphoreType.DMA,
        ],
    )


NBH = NB // NC


def _sc_deg_body(dstp, z128, ones_hbm, d0_out, d1_out, dega, dst_v, ones_v,
                 sem):
    cid = lax.axis_index("c")
    sid = lax.axis_index("s")

    _stripe_copy(sid, z128, dega)
    pltpu.sync_copy(ones_hbm, ones_v)
    plsc.subcore_barrier()

    def blk_body(i, carry):
        blk = (sid * NC + cid) * NBH + i
        pltpu.sync_copy(dstp.at[blk], dst_v)
        for j in range(8):
            pltpu.sync_copy(ones_v, dega.at[dst_v.at[j]], add=True)
        return carry
    lax.fori_loop(0, NBH, blk_body, 0)

    plsc.subcore_barrier()

    @pl.when(cid == 0)
    def _():
        _stripe_copy(sid, dega, d0_out)

    @pl.when(cid == 1)
    def _():
        _stripe_copy(sid, dega, d1_out)


def _make_sc_deg():
    f32 = jnp.float32
    return pl.kernel(
        _sc_deg_body,
        out_type=[
            jax.ShapeDtypeStruct((N_NODES, DH), f32),
            jax.ShapeDtypeStruct((N_NODES, DH), f32),
        ],
        mesh=_mesh(),
        scratch_types=[
            pltpu.VMEM_SHARED((N_ACC, DH), f32),
            pltpu.VMEM((8, 128), jnp.int32),
            pltpu.VMEM((128, DH), f32),
            pltpu.SemaphoreType.DMA,
        ],
    )


def _sc_agg_split_body(y, srcp, dstp, z128, p0_out, p1_out,
                       acc, src_v, dst_v, rows_v, gsem, ssem0, ssem1):
    cid = lax.axis_index("c")
    sid = lax.axis_index("s")
    ssems = (ssem0, ssem1)

    _stripe_copy(sid, z128, acc)
    plsc.subcore_barrier()

    def blk_body(i, carry):
        blk = (sid * NC + cid) * NBH + i
        pltpu.sync_copy(srcp.at[blk], src_v)
        pltpu.sync_copy(dstp.at[blk], dst_v)
        gd = _fire_gather(y, src_v, rows_v, 0, 0, gsem)
        scat = [None] * 8
        for j in range(8):
            for g in gd:
                g.wait()
            scat[j] = pltpu.async_copy(
                rows_v.at[j % 2], acc.at[dst_v.at[j]], ssems[j % 2],
                add=True,
            )
            if j < 7:
                if j >= 1:
                    scat[j - 1].wait()
                gd = _fire_gather(y, src_v, rows_v, j + 1, (j + 1) % 2,
                                  gsem)
        scat[6].wait()
        scat[7].wait()
        return carry
    lax.fori_loop(0, NBH, blk_body, 0)

    plsc.subcore_barrier()

    @pl.when(cid == 0)
    def _():
        _stripe_copy(sid, acc, p0_out)

    @pl.when(cid == 1)
    def _():
        _stripe_copy(sid, acc, p1_out)


def _make_sc_agg_split():
    f32 = jnp.float32
    return pl.kernel(
        _sc_agg_split_body,
        out_type=[
            jax.ShapeDtypeStruct((N_NODES, DH), f32),
            jax.ShapeDtypeStruct((N_NODES, DH), f32),
        ],
        mesh=_mesh(),
        scratch_types=[
            pltpu.VMEM_SHARED((N_ACC, DH), f32),
            pltpu.VMEM((8, 128), jnp.int32),
            pltpu.VMEM((8, 128), jnp.int32),
            pltpu.VMEM((2, 128, DH), f32),
            pltpu.SemaphoreType.DMA,
            pltpu.SemaphoreType.DMA,
            pltpu.SemaphoreType.DMA,
        ],
    )


def _dot(a, b):
    return jnp.dot(a, b, preferred_element_type=jnp.float32)


def _sage_block(h0, h1, s0, s1, d0, d1, ws, wn):
    h = jnp.concatenate([h0[...], h1[...]], axis=1)
    s = jnp.concatenate([s0[...], s1[...]], axis=1)
    d = jnp.maximum(d0[:, 0:1] + d1[:, 0:1], 1.0)
    return _dot(h, ws[...]) + _dot(s / d, wn[...])


def _tc_layer0_body(h0, h1, s0, s1, d0, d1, ws, wn, o0, o1):
    out = jnp.maximum(_sage_block(h0, h1, s0, s1, d0, d1, ws, wn), 0.0)
    o0[...] = out[:, :DH]
    o1[...] = out[:, DH:]


def _tc_layer1_body(h0, h1, s0, s1, d0, d1, ws, wn, ws2, wn2, y2, z2):
    h2 = jnp.maximum(_sage_block(h0, h1, s0, s1, d0, d1, ws, wn), 0.0)
    y2[...] = _dot(h2, wn2[...])
    z2[...] = _dot(h2, ws2[...])


def _tc_final_body(z2, p0, p1, d0, d1, o):
    d = jnp.maximum(d0[:, 0:1] + d1[:, 0:1], 1.0)
    o[...] = z2[...] + (p0[...] + p1[...]) / d


BR = 400


def _row_spec(w):
    return pl.BlockSpec((BR, w), lambda i: (i, 0))


def _full_spec(a, b):
    return pl.BlockSpec((a, b), lambda i: (0, 0))


def _tc_layer0(h0, h1, s0, s1, d0, d1, ws_t, wn_t):
    return pl.pallas_call(
        _tc_layer0_body,
        grid=(N_NODES // BR,),
        in_specs=[_row_spec(DH)] * 6 + [
            _full_spec(D_HID, D_HID), _full_spec(D_HID, D_HID),
        ],
        out_specs=[_row_spec(DH), _row_spec(DH)],
        out_shape=[
            jax.ShapeDtypeStruct((N_NODES, DH), jnp.float32),
            jax.ShapeDtypeStruct((N_NODES, DH), jnp.float32),
        ],
    )(h0, h1, s0, s1, d0, d1, ws_t, wn_t)


def _tc_layer1(h0, h1, s0, s1, d0, d1, ws_t, wn_t, ws2_t, wn2_t):
    return pl.pallas_call(
        _tc_layer1_body,
        grid=(N_NODES // BR,),
        in_specs=[_row_spec(DH)] * 6 + [
            _full_spec(D_HID, D_HID), _full_spec(D_HID, D_HID),
            _full_spec(D_HID, DH), _full_spec(D_HID, DH),
        ],
        out_specs=[_row_spec(DH), _row_spec(DH)],
        out_shape=[
            jax.ShapeDtypeStruct((N_NODES, DH), jnp.float32),
            jax.ShapeDtypeStruct((N_NODES, DH), jnp.float32),
        ],
    )(h0, h1, s0, s1, d0, d1, ws_t, wn_t, ws2_t, wn2_t)


def _tc_final(z2, p0, p1, d0, d1):
    return pl.pallas_call(
        _tc_final_body,
        grid=(N_NODES // BR,),
        in_specs=[_row_spec(DH)] * 5,
        out_specs=_row_spec(DH),
        out_shape=jax.ShapeDtypeStruct((N_NODES, DH), jnp.float32),
    )(z2, p0, p1, d0, d1)


def kernel(x, edge_index, W_self_0, W_neigh_0, W_self_1, W_neigh_1,
           W_self_2, W_neigh_2):
    f32 = jnp.float32
    x0 = x[:, :DH]
    x1 = x[:, DH:]
    src = edge_index[0]
    dst = edge_index[1]
    pad = E_PAD - N_EDGES
    srcp = jnp.concatenate([src, jnp.zeros((pad,), jnp.int32)]).reshape(
        E_PAD // BLK, 8, 128)
    dstp = jnp.concatenate([dst, jnp.full((pad,), N_NODES, jnp.int32)]
                           ).reshape(E_PAD // BLK, 8, 128)
    z128 = jnp.zeros((N_NODES, DH), f32)
    ones = jnp.ones((128, DH), f32)

    sc_agg = _make_sc_agg()
    d0, d1 = _make_sc_deg()(dstp, z128, ones)

    w_pad = 128 - N_CLASSES
    ws2 = jnp.pad(W_self_2.T, ((0, 0), (0, w_pad)))
    wn2 = jnp.pad(W_neigh_2.T, ((0, 0), (0, w_pad)))

    s0, s1 = sc_agg(x0, x1, srcp, dstp, z128)
    h0, h1 = _tc_layer0(x0, x1, s0, s1, d0, d1, W_self_0.T, W_neigh_0.T)
    s0, s1 = sc_agg(h0, h1, srcp, dstp, z128)
    y2, z2 = _tc_layer1(h0, h1, s0, s1, d0, d1, W_self_1.T, W_neigh_1.T,
                        ws2, wn2)
    p0, p1 = _make_sc_agg_split()(y2, srcp, dstp, z128)
    out = _tc_final(z2, p0, p1, d0, d1)
    return out[:, :N_CLASSES]
